# baseline (device time: 123212 ns/iter reference)
import jax
import jax.numpy as jnp
from jax import lax
from jax.experimental import pallas as pl
from jax.experimental.pallas import tpu as pltpu

Z = 4


def kernel(ids, E):
    v_per, d = E.shape
    t = ids.shape[0]

    my_z = lax.axis_index("z")
    local = ids - my_z * v_per
    mask = (local >= 0) & (local < v_per)
    safe = jnp.where(mask, local, 0)
    part = jnp.take(E, safe, axis=0, mode="clip")
    part = jnp.where(mask[:, None], part, 0.0).astype(jnp.bfloat16)

    def body(part_ref, out_ref, comm_ref, send_sems, recv_sems):
        x = lax.axis_index("x")
        y = lax.axis_index("y")
        z = lax.axis_index("z")
        left = (z - 1) % Z
        right = (z + 1) % Z

        barrier_sem = pltpu.get_barrier_semaphore()
        for nbr in [left, right]:
            pl.semaphore_signal(
                barrier_sem, inc=1,
                device_id=(x, y, nbr), device_id_type=pl.DeviceIdType.MESH,
            )
        pl.semaphore_wait(barrier_sem, 2)

        out_ref[...] = part_ref[...].astype(jnp.float32)
        comm_ref[0] = part_ref[...]

        for h in range(Z - 1):
            rdma = pltpu.make_async_remote_copy(
                src_ref=comm_ref.at[h],
                dst_ref=comm_ref.at[h + 1],
                send_sem=send_sems.at[h],
                recv_sem=recv_sems.at[h],
                device_id=(x, y, right),
                device_id_type=pl.DeviceIdType.MESH,
            )
            rdma.start()
            rdma.wait()
            out_ref[...] += comm_ref[h + 1].astype(jnp.float32)

    return pl.pallas_call(
        body,
        out_shape=jax.ShapeDtypeStruct((t, d), jnp.float32),
        in_specs=[pl.BlockSpec(memory_space=pltpu.VMEM)],
        out_specs=pl.BlockSpec(memory_space=pltpu.VMEM),
        scratch_shapes=[
            pltpu.VMEM((Z, t, d), jnp.bfloat16),
            pltpu.SemaphoreType.DMA((Z - 1,)),
            pltpu.SemaphoreType.DMA((Z - 1,)),
        ],
        compiler_params=pltpu.CompilerParams(collective_id=0),
    )(part)


# device time: 69345 ns/iter; 1.7768x vs baseline; 1.7768x over previous
import jax
import jax.numpy as jnp
from jax import lax
from jax.experimental import pallas as pl
from jax.experimental.pallas import tpu as pltpu

Z = 4
NBLK = 4


def kernel(ids, E):
    v_per, d = E.shape
    t = ids.shape[0]
    tb = t // NBLK

    E_bf = E.astype(jnp.bfloat16)
    ids_blocks = ids.reshape(NBLK, tb, 1)

    def body(ids_ref, e_ref, out_ref, pbuf, red, comm, ssems, rsems):
        x = lax.axis_index("x")
        y = lax.axis_index("y")
        z = lax.axis_index("z")
        left = (z - 1) % Z
        right = (z + 1) % Z

        barrier_sem = pltpu.get_barrier_semaphore()
        for nbr in [left, right]:
            pl.semaphore_signal(
                barrier_sem, inc=1,
                device_id=(x, y, nbr), device_id_type=pl.DeviceIdType.MESH,
            )
        pl.semaphore_wait(barrier_sem, 2)

        col = lax.broadcasted_iota(jnp.int32, (tb, v_per), 1)

        def compute_partial(k, blk):
            ids_b = ids_ref[pl.ds(blk, 1)].reshape(tb, 1)
            oh = (col == ids_b - z * v_per).astype(jnp.bfloat16)
            acc = lax.dot_general(
                oh, e_ref[...], (((1,), (0,)), ((), ())),
                preferred_element_type=jnp.float32,
            )
            pbuf[k] = acc.astype(jnp.bfloat16)

        def step_rdma(s, src):
            return pltpu.make_async_remote_copy(
                src_ref=src,
                dst_ref=comm.at[s],
                send_sem=ssems.at[s],
                recv_sem=rsems.at[s],
                device_id=(x, y, right),
                device_id_type=pl.DeviceIdType.MESH,
            )

        compute_partial(0, z % NBLK)
        rdmas = []

        for s in range(Z - 1):
            rd = step_rdma(s, pbuf.at[0] if s == 0 else red.at[s - 1])
            rd.start()
            rdmas.append(rd)
            compute_partial(s + 1, (z - s - 1) % NBLK)
            rd.wait_recv()
            red[s] = comm[s] + pbuf[s + 1]

        for s in range(Z - 1):
            ag = step_rdma(Z - 1 + s, red.at[Z - 2] if s == 0 else comm.at[Z - 2 + s])
            ag.start()
            rdmas.append(ag)
            blk = (z + 1 - s) % NBLK
            src = red[Z - 2] if s == 0 else comm[Z - 2 + s]
            out_ref[pl.ds(blk * tb, tb), :] = src.astype(jnp.float32)
            ag.wait_recv()

        out_ref[pl.ds(((z + 2) % NBLK) * tb, tb), :] = comm[2 * Z - 3].astype(
            jnp.float32
        )

        for rd in rdmas:
            rd.wait_send()

    return pl.pallas_call(
        body,
        out_shape=jax.ShapeDtypeStruct((t, d), jnp.float32),
        in_specs=[
            pl.BlockSpec(memory_space=pltpu.VMEM),
            pl.BlockSpec(memory_space=pltpu.VMEM),
        ],
        out_specs=pl.BlockSpec(memory_space=pltpu.VMEM),
        scratch_shapes=[
            pltpu.VMEM((NBLK, tb, d), jnp.bfloat16),
            pltpu.VMEM((Z - 1, tb, d), jnp.bfloat16),
            pltpu.VMEM((2 * Z - 2, tb, d), jnp.bfloat16),
            pltpu.SemaphoreType.DMA((2 * Z - 2,)),
            pltpu.SemaphoreType.DMA((2 * Z - 2,)),
        ],
        compiler_params=pltpu.CompilerParams(collective_id=0),
    )(ids_blocks, E_bf)
